# baseline (speedup 1.0000x reference)
import functools

import jax
import jax.numpy as jnp
from jax import lax
from jax.experimental import pallas as pl
from jax.experimental.pallas import tpu as pltpu
from jax.experimental.pallas import tpu_sc as plsc

B = 4096
C = 30
CPAD = 32
EDIM = 64
NT = 113
NW = 32
RPW = B // NW
CB = 4
CBI = CB * CPAD
NCHUNK = RPW // CB
L = 16
V = 1000000
TROWS = V // 2
W = 384
NPAN = V // W
NPPW = (NPAN + NW - 1) // NW


def _tp_body(tabt_hbm, tail_hbm, out_hbm, in_v, out_v, semi0, semi1,
             semo0, semo1):
    wid = lax.axis_index("s") * 2 + lax.axis_index("c")

    @pl.when(wid == 0)
    def _():
        pltpu.sync_copy(
            tail_hbm, out_hbm.at[pl.ds(pl.multiple_of(NPAN * (W // 2), 8),
                                       V // 2 - NPAN * (W // 2))])
    iota = lax.iota(jnp.int32, L)
    semi = (semi0, semi1)
    semo = (semo0, semo1)

    def issue_in(i, p):
        pid = i * NW + wid
        off = pl.multiple_of(pid * W, 128)
        pltpu.async_copy(tabt_hbm.at[:, pl.ds(off, W)], in_v.at[p], semi[p])

    def wait_in(i, p):
        pltpu.make_async_copy(tabt_hbm.at[:, pl.ds(0, W)],
                              in_v.at[p], semi[p]).wait()

    def transpose_panel(inp, outp, ne):
        def eg_body(eg, carry):
            ebase = eg * L
            ev = ebase + iota
            r0 = ev >> 1
            c0 = (ev & 1) * EDIM

            def d_body(d, carry2):
                dv = jnp.broadcast_to(d, (L,))
                v = plsc.load_gather(inp, [dv, ev])
                plsc.store_scatter(outp, [r0, c0 + dv], v)
                return carry2

            lax.fori_loop(0, EDIM, d_body, 0, unroll=8)
            return carry

        lax.fori_loop(0, ne // L, eg_body, 0)

    def issue_out(i, p):
        pid = i * NW + wid
        off = pl.multiple_of(pid * (W // 2), 8)
        pltpu.async_copy(out_v.at[p], out_hbm.at[pl.ds(off, W // 2)],
                         semo[p])

    def wait_out(i, p):
        pltpu.make_async_copy(out_v.at[p], out_hbm.at[pl.ds(0, W // 2)],
                              semo[p]).wait()

    def guarded_issue_in(i, p):
        @pl.when(i * NW + wid < NPAN)
        def _():
            issue_in(i, p)

    guarded_issue_in(0, 0)
    guarded_issue_in(1, 1)

    def pair_body(ii, carry):
        for p in (0, 1):
            i = 2 * ii + p

            @pl.when((i >= 2) & ((i - 2) * NW + wid < NPAN))
            def _():
                wait_out(i - 2, p)

            @pl.when(i * NW + wid < NPAN)
            def _():
                wait_in(i, p)
                transpose_panel(in_v.at[p], out_v.at[p], W)
                issue_out(i, p)

            guarded_issue_in(i + 2, p)
        return carry

    lax.fori_loop(0, NPPW // 2, pair_body, 0)

    for i in (NPPW - 2, NPPW - 1):
        @pl.when(i * NW + wid < NPAN)
        def _():
            wait_out(i, i & 1)


@functools.partial(
    pl.kernel,
    mesh=plsc.VectorSubcoreMesh(core_axis_name="c", subcore_axis_name="s"),
    compiler_params=pltpu.CompilerParams(needs_layout_passes=False),
    out_type=jax.ShapeDtypeStruct((TROWS, 2 * EDIM), jnp.float32),
    scratch_types=[
        pltpu.VMEM((2, EDIM, W), jnp.float32),
        pltpu.VMEM((2, W // 2, 2 * EDIM), jnp.float32),
        pltpu.SemaphoreType.DMA,
        pltpu.SemaphoreType.DMA,
        pltpu.SemaphoreType.DMA,
        pltpu.SemaphoreType.DMA,
    ],
)
def _tp_table(tabt_hbm, tail_hbm, out_hbm, *scratch):
    _tp_body(tabt_hbm, tail_hbm, out_hbm, *scratch)


def _sc_body(idx_hbm, par_hbm, ctx_hbm, tab_hbm, sco_hbm, prb_hbm,
             idx_v, par_v, ctx_v, emb_v, sco_v, prb_v, sem0, sem1):
    wid = lax.axis_index("s") * 2 + lax.axis_index("c")
    row0 = wid * RPW
    nidx = NCHUNK * CBI
    ibase = pl.multiple_of(wid * nidx, nidx)

    pltpu.sync_copy(idx_hbm.at[pl.ds(ibase, nidx)], idx_v)
    pltpu.sync_copy(par_hbm.at[pl.ds(ibase, nidx)], par_v)
    pltpu.sync_copy(
        ctx_hbm.at[pl.ds(pl.multiple_of(wid * (RPW // 2), RPW // 2),
                         RPW // 2)], ctx_v)

    iota = lax.iota(jnp.int32, L)
    sems = (sem0, sem1)

    def issue(k, p):
        return pltpu.async_copy(
            tab_hbm.at[idx_v.at[pl.ds(k * CBI, CBI)]], emb_v.at[p], sems[p])

    def compute_chunk(k, embp):
        def row_body(r, carry):
            row = k * CB + r
            coff = (row & 1) * EDIM
            rowv = jnp.broadcast_to(row >> 1, (L,))
            rows0 = r * CPAD + iota
            rows1 = rows0 + L
            coffv = jnp.broadcast_to(coff, (L,))
            par0 = par_v[pl.ds(row * CPAD, L)] - coffv
            par1 = par_v[pl.ds(row * CPAD + L, L)] - coffv

            def d_body(d, accs):
                a0, a1, b0, b1 = accs
                dv = jnp.broadcast_to(d + coff, (L,))
                cb = plsc.load_gather(ctx_v, [rowv, dv])
                e0 = plsc.load_gather(embp, [rows0, par0 + dv])
                e1 = plsc.load_gather(embp, [rows1, par1 + dv])
                dv2 = dv + 1
                cb2 = plsc.load_gather(ctx_v, [rowv, dv2])
                e0b = plsc.load_gather(embp, [rows0, par0 + dv2])
                e1b = plsc.load_gather(embp, [rows1, par1 + dv2])
                return (a0 + cb * e0, a1 + cb * e1,
                        b0 + cb2 * e0b, b1 + cb2 * e1b)

            z = jnp.zeros((L,), jnp.float32)
            a0, a1, b0, b1 = lax.fori_loop(0, EDIM // 2,
                                           lambda i, acc: d_body(2 * i, acc),
                                           (z, z, z, z), unroll=8)
            s0 = a0 + b0
            s1 = a1 + b1

            m = jnp.maximum(jnp.max(s0), jnp.max(s1))
            mb = jnp.broadcast_to(m, (L,))
            e0 = jnp.exp(s0 - mb)
            e1 = jnp.where(iota < (C - L), jnp.exp(s1 - mb),
                           jnp.zeros((L,), jnp.float32))
            t = jnp.sum(e0) + jnp.sum(e1)
            invb = jnp.ones((L,), jnp.float32) / jnp.broadcast_to(t, (L,))
            base = row * CPAD
            sco_v[pl.ds(base, L)] = s0
            sco_v[pl.ds(base + L, L)] = s1
            prb_v[pl.ds(base, L)] = e0 * invb
            prb_v[pl.ds(base + L, L)] = e1 * invb
            return carry

        lax.fori_loop(0, CB, row_body, 0)

    h = [issue(0, 0), issue(1, 1)]
    for k in range(NCHUNK):
        p = k & 1
        h[p].wait()
        compute_chunk(k, emb_v.at[p])
        if k + 2 < NCHUNK:
            h[p] = issue(k + 2, p)

    obase = pl.multiple_of(row0 * CPAD, RPW * CPAD)
    pltpu.sync_copy(sco_v, sco_hbm.at[pl.ds(obase, RPW * CPAD)])
    pltpu.sync_copy(prb_v, prb_hbm.at[pl.ds(obase, RPW * CPAD)])


@functools.partial(
    pl.kernel,
    mesh=plsc.VectorSubcoreMesh(core_axis_name="c", subcore_axis_name="s"),
    compiler_params=pltpu.CompilerParams(needs_layout_passes=False),
    out_type=[
        jax.ShapeDtypeStruct((B * CPAD,), jnp.float32),
        jax.ShapeDtypeStruct((B * CPAD,), jnp.float32),
    ],
    scratch_types=[
        pltpu.VMEM((NCHUNK * CBI,), jnp.int32),
        pltpu.VMEM((NCHUNK * CBI,), jnp.int32),
        pltpu.VMEM((RPW // 2, 2 * EDIM), jnp.float32),
        pltpu.VMEM((2, CBI, 2 * EDIM), jnp.float32),
        pltpu.VMEM((RPW * CPAD,), jnp.float32),
        pltpu.VMEM((RPW * CPAD,), jnp.float32),
        pltpu.SemaphoreType.DMA,
        pltpu.SemaphoreType.DMA,
    ],
)
def _sc_scores(idx_hbm, par_hbm, ctx_hbm, tab_hbm, sco_hbm, prb_hbm,
               *scratch):
    _sc_body(idx_hbm, par_hbm, ctx_hbm, tab_hbm, sco_hbm, prb_hbm, *scratch)


def _tc_body(ctx_ref, w_ref, b_ref, o_ref):
    y = jnp.dot(ctx_ref[...], w_ref[...],
                preferred_element_type=jnp.float32) + b_ref[...]
    o_ref[...] = jax.nn.sigmoid(y)


def _mentype(ctx, w, b2d):
    return pl.pallas_call(
        _tc_body,
        out_shape=jax.ShapeDtypeStruct((B, NT), jnp.float32),
    )(ctx, w, b2d)


def kernel(leftb, rightb, leftlens, rightlens, docb, wididxsb,
           entity_table, context_encoded, type_W, type_b):
    idx_pad = jnp.concatenate([wididxsb, wididxsb[:, C - 2:]], axis=1)
    idx1d = (idx_pad >> 1).reshape(-1)
    par1d = ((idx_pad & 1) * EDIM).reshape(-1)
    ntail = V // 2 - NPAN * (W // 2)
    tail = entity_table[NPAN * W:].reshape(ntail, 2 * EDIM)
    tab2 = _tp_table(entity_table.T, tail)
    ctx2 = context_encoded.reshape(B // 2, 2 * EDIM)
    sco_f, prb_f = _sc_scores(idx1d, par1d, ctx2, tab2)
    sco = sco_f.reshape(B, CPAD)[:, :C]
    prb = prb_f.reshape(B, CPAD)[:, :C]
    ment = _mentype(context_encoded, type_W, type_b.reshape(1, NT))
    return sco, prb, ment


# speedup vs baseline: 2.4253x; 2.4253x over previous
import functools

import jax
import jax.numpy as jnp
from jax import lax
from jax.experimental import pallas as pl
from jax.experimental.pallas import tpu as pltpu
from jax.experimental.pallas import tpu_sc as plsc

B = 4096
C = 30
CPAD = 32
EDIM = 64
NT = 113
NW = 32
RPW = B // NW
CBI = CPAD
NCHUNK = RPW
L = 16


def _sc_body(base_hbm, sub_hbm, ctx_hbm, tab_hbm, sco_hbm, prb_hbm,
             base_v, sub_v, ctx_v, emb_v, sco_v, prb_v, semg0, semg1):
    wid = lax.axis_index("s") * 2 + lax.axis_index("c")
    row0 = wid * RPW
    nidx = NCHUNK * CBI
    ibase = pl.multiple_of(wid * nidx, nidx)

    pltpu.sync_copy(base_hbm.at[pl.ds(ibase, nidx)], base_v)
    pltpu.sync_copy(sub_hbm.at[pl.ds(ibase, nidx)], sub_v)
    pltpu.sync_copy(
        ctx_hbm.at[pl.ds(pl.multiple_of(wid * (RPW // 2), RPW // 2),
                         RPW // 2)], ctx_v)

    iota = lax.iota(jnp.int32, L)
    iota8 = iota * 8
    semg = (semg0, semg1)

    def issue_gathers(k, p):
        for g in range(2):
            vb = base_v[pl.ds(k * CBI + g * L, L)]
            for j in range(L):
                t = vb[j]
                s = g * L + j
                pltpu.async_copy(
                    tab_hbm.at[pl.ds(pl.multiple_of(t, 8), 8)],
                    emb_v.at[p, pl.ds(s * 8, 8)], semg[p])

    def wait_gathers(p):
        pltpu.make_async_copy(tab_hbm.at[pl.ds(0, CBI * 8)], emb_v.at[p],
                              semg[p]).wait()

    def compute_chunk(row, embp):
        coff = (row & 1) * EDIM
        rowv = jnp.broadcast_to(row >> 1, (L,))
        coffv = jnp.broadcast_to(coff, (L,))
        rows0 = iota8 + sub_v[pl.ds(row * CPAD, L)]
        rows1 = iota8 + jnp.broadcast_to(L * 8, (L,)) \
            + sub_v[pl.ds(row * CPAD + L, L)]

        def d_body(d, accs):
            a0, a1, b0, b1 = accs
            dc = jnp.broadcast_to(d + coff, (L,))
            dv = dc - coffv
            cb = plsc.load_gather(ctx_v, [rowv, dc])
            e0 = plsc.load_gather(embp, [rows0, dv])
            e1 = plsc.load_gather(embp, [rows1, dv])
            dc2 = dc + 1
            dv2 = dv + 1
            cb2 = plsc.load_gather(ctx_v, [rowv, dc2])
            e0b = plsc.load_gather(embp, [rows0, dv2])
            e1b = plsc.load_gather(embp, [rows1, dv2])
            return (a0 + cb * e0, a1 + cb * e1,
                    b0 + cb2 * e0b, b1 + cb2 * e1b)

        z = jnp.zeros((L,), jnp.float32)
        a0, a1, b0, b1 = lax.fori_loop(0, EDIM // 2,
                                       lambda i, acc: d_body(2 * i, acc),
                                       (z, z, z, z), unroll=8)
        s0 = a0 + b0
        s1 = a1 + b1

        m = jnp.maximum(jnp.max(s0), jnp.max(s1))
        mb = jnp.broadcast_to(m, (L,))
        e0 = jnp.exp(s0 - mb)
        e1 = jnp.where(iota < (C - L), jnp.exp(s1 - mb),
                       jnp.zeros((L,), jnp.float32))
        t = jnp.sum(e0) + jnp.sum(e1)
        invb = jnp.ones((L,), jnp.float32) / jnp.broadcast_to(t, (L,))
        base = row * CPAD
        sco_v[pl.ds(base, L)] = s0
        sco_v[pl.ds(base + L, L)] = s1
        prb_v[pl.ds(base, L)] = e0 * invb
        prb_v[pl.ds(base + L, L)] = e1 * invb

    issue_gathers(0, 0)
    issue_gathers(1, 1)

    def pair_body(kk, carry):
        for p in (0, 1):
            k = 2 * kk + p
            wait_gathers(p)
            compute_chunk(k, emb_v.at[p])

            @pl.when(k + 2 < NCHUNK)
            def _():
                issue_gathers(k + 2, p)
        return carry

    lax.fori_loop(0, NCHUNK // 2, pair_body, 0)

    obase = pl.multiple_of(row0 * CPAD, RPW * CPAD)
    pltpu.sync_copy(sco_v, sco_hbm.at[pl.ds(obase, RPW * CPAD)])
    pltpu.sync_copy(prb_v, prb_hbm.at[pl.ds(obase, RPW * CPAD)])


@functools.partial(
    pl.kernel,
    mesh=plsc.VectorSubcoreMesh(core_axis_name="c", subcore_axis_name="s"),
    compiler_params=pltpu.CompilerParams(needs_layout_passes=False),
    out_type=[
        jax.ShapeDtypeStruct((B * CPAD,), jnp.float32),
        jax.ShapeDtypeStruct((B * CPAD,), jnp.float32),
    ],
    scratch_types=[
        pltpu.VMEM((NCHUNK * CBI,), jnp.int32),
        pltpu.VMEM((NCHUNK * CBI,), jnp.int32),
        pltpu.VMEM((RPW // 2, 2 * EDIM), jnp.float32),
        pltpu.VMEM((2, CBI * 8, EDIM), jnp.float32),
        pltpu.VMEM((RPW * CPAD,), jnp.float32),
        pltpu.VMEM((RPW * CPAD,), jnp.float32),
        pltpu.SemaphoreType.DMA,
        pltpu.SemaphoreType.DMA,
    ],
)
def _sc_scores(base_hbm, sub_hbm, ctx_hbm, tab_hbm, sco_hbm, prb_hbm,
               *scratch):
    _sc_body(base_hbm, sub_hbm, ctx_hbm, tab_hbm, sco_hbm, prb_hbm, *scratch)


def _tc_body(ctx_ref, w_ref, b_ref, o_ref):
    y = jnp.dot(ctx_ref[...], w_ref[...],
                preferred_element_type=jnp.float32) + b_ref[...]
    o_ref[...] = jax.nn.sigmoid(y)


def _mentype(ctx, w, b2d):
    return pl.pallas_call(
        _tc_body,
        out_shape=jax.ShapeDtypeStruct((B, NT), jnp.float32),
    )(ctx, w, b2d)


def kernel(leftb, rightb, leftlens, rightlens, docb, wididxsb,
           entity_table, context_encoded, type_W, type_b):
    idx_pad = jnp.concatenate([wididxsb, wididxsb[:, C - 2:]], axis=1)
    base1d = (idx_pad & ~7).reshape(-1)
    sub1d = (idx_pad & 7).reshape(-1)
    ctx2 = context_encoded.reshape(B // 2, 2 * EDIM)
    sco_f, prb_f = _sc_scores(base1d, sub1d, ctx2, entity_table)
    sco = sco_f.reshape(B, CPAD)[:, :C]
    prb = prb_f.reshape(B, CPAD)[:, :C]
    ment = _mentype(context_encoded, type_W, type_b.reshape(1, NT))
    return sco, prb, ment
